# baseline (speedup 1.0000x reference)
import functools

import jax
import jax.numpy as jnp
import numpy as np
from jax import lax
from jax.experimental import pallas as pl
from jax.experimental.pallas import tpu as pltpu
from jax.experimental.pallas import tpu_sc as plsc
from jax.sharding import Mesh, PartitionSpec as P

D = 1024
H = 4096
NNZ = 8192
W = 4
L = 16
NWORKERS = 32


def _make_body(spt):
  def _body(x_hbm, ur_hbm, uc_hbm, uv_hbm, ub_hbm,
            gr_hbm, gc_hbm, gv_hbm, gb_hbm,
            dr_hbm, dc_hbm, dv_hbm, db_hbm,
            out_hbm,
            x_v, up_acc, gate_acc, down_acc,
            ur_v, uc_v, uv_v, gr_v, gc_v, gv_v, dr_v, dc_v, dv_v,
            ub_v, gb_v, db_v):
    wid = lax.axis_index("s") * 2 + lax.axis_index("c")

    pltpu.sync_copy(ub_hbm, ub_v)
    pltpu.sync_copy(gb_hbm, gb_v)
    pltpu.sync_copy(db_hbm, db_v)
    pltpu.sync_copy(ur_hbm, ur_v)
    pltpu.sync_copy(uc_hbm, uc_v)
    pltpu.sync_copy(uv_hbm, uv_v)
    pltpu.sync_copy(gr_hbm, gr_v)
    pltpu.sync_copy(gc_hbm, gc_v)
    pltpu.sync_copy(gv_hbm, gv_v)
    pltpu.sync_copy(dr_hbm, dr_v)
    pltpu.sync_copy(dc_hbm, dc_v)
    pltpu.sync_copy(dv_hbm, dv_v)

    def spmv(row_v, col_v, val_v, src_ref, src_stride, acc_ref, acc_stride):
      @plsc.parallel_loop(0, NNZ // L, unroll=4)
      def _(g):
        o = g * L
        rows = row_v[pl.ds(o, L)]
        cols = col_v[pl.ds(o, L)]
        vals = val_v[pl.ds(o, L)]
        for j in range(W):
          xg = plsc.load_gather(src_ref.at[pl.ds(j * src_stride, src_stride)],
                                [cols])
          plsc.addupdate_scatter(acc_ref.at[pl.ds(j * acc_stride, acc_stride)],
                                 [rows], xg * vals)

    def slice_body(i, carry):
      sl = wid * spt + i
      xoff = sl * (W * D)
      pltpu.sync_copy(x_hbm.at[pl.ds(xoff, W * D)], x_v)

      @plsc.parallel_loop(0, H // L, unroll=4)
      def _(k):
        o = k * L
        ub = ub_v[pl.ds(o, L)]
        gb = gb_v[pl.ds(o, L)]
        for j in range(W):
          up_acc[pl.ds(j * H + o, L)] = ub
          gate_acc[pl.ds(j * H + o, L)] = gb

      @plsc.parallel_loop(0, D // L, unroll=4)
      def _(k):
        o = k * L
        db = db_v[pl.ds(o, L)]
        for j in range(W):
          down_acc[pl.ds(j * D + o, L)] = db + x_v[pl.ds(j * D + o, L)]

      spmv(ur_v, uc_v, uv_v, x_v, D, up_acc, H)
      spmv(gr_v, gc_v, gv_v, x_v, D, gate_acc, H)

      @plsc.parallel_loop(0, (W * H) // L, unroll=4)
      def _(k):
        o = k * L
        u = up_acc[pl.ds(o, L)]
        g = gate_acc[pl.ds(o, L)]
        up_acc[pl.ds(o, L)] = (u / (1.0 + jnp.exp(-u))) * g

      spmv(dr_v, dc_v, dv_v, up_acc, H, down_acc, D)

      pltpu.sync_copy(down_acc, out_hbm.at[pl.ds(xoff, W * D)])
      return carry

    lax.fori_loop(0, spt, slice_body, 0)

  return _body


@functools.cache
def _make_sswiglu(b_local):
  spt = b_local // (W * NWORKERS)
  assert spt * W * NWORKERS == b_local
  return functools.partial(
      pl.kernel,
      mesh=plsc.VectorSubcoreMesh(core_axis_name="c", subcore_axis_name="s"),
      out_type=jax.ShapeDtypeStruct((b_local * D,), jnp.float32),
      compiler_params=pltpu.CompilerParams(needs_layout_passes=False),
      scratch_types=[
          pltpu.VMEM((W * D,), jnp.float32),
          pltpu.VMEM((W * H,), jnp.float32),
          pltpu.VMEM((W * H,), jnp.float32),
          pltpu.VMEM((W * D,), jnp.float32),
          pltpu.VMEM((NNZ,), jnp.int32),
          pltpu.VMEM((NNZ,), jnp.int32),
          pltpu.VMEM((NNZ,), jnp.float32),
          pltpu.VMEM((NNZ,), jnp.int32),
          pltpu.VMEM((NNZ,), jnp.int32),
          pltpu.VMEM((NNZ,), jnp.float32),
          pltpu.VMEM((NNZ,), jnp.int32),
          pltpu.VMEM((NNZ,), jnp.int32),
          pltpu.VMEM((NNZ,), jnp.float32),
          pltpu.VMEM((H,), jnp.float32),
          pltpu.VMEM((H,), jnp.float32),
          pltpu.VMEM((D,), jnp.float32),
      ],
  )(_make_body(spt))


def kernel(x, up_row, up_col, up_vals, up_bias,
           gate_row, gate_col, gate_vals, gate_bias,
           down_row, down_col, down_vals, down_bias):
  shape = x.shape
  b_total = x.size // D
  devs = jax.devices()
  ndev = 2 if (len(devs) >= 2 and b_total % 2 == 0) else 1
  x_flat = x.reshape(-1)

  def run(xs, ur, uc, uv, ub, gr, gc, gv, gb, dr, dc, dv, db):
    return _make_sswiglu(xs.size // D)(xs, ur, uc, uv, ub,
                                       gr, gc, gv, gb, dr, dc, dv, db)

  if ndev == 1:
    out = run(x_flat, up_row, up_col, up_vals, up_bias,
              gate_row, gate_col, gate_vals, gate_bias,
              down_row, down_col, down_vals, down_bias)
  else:
    mesh = Mesh(np.array(devs[:ndev]), ("b",))
    rep = P()
    f = jax.shard_map(
        run, mesh=mesh, check_vma=False,
        in_specs=(P("b"), rep, rep, rep, rep, rep, rep, rep, rep,
                  rep, rep, rep, rep),
        out_specs=P("b"))
    out = f(x_flat, up_row, up_col, up_vals, up_bias,
            gate_row, gate_col, gate_vals, gate_bias,
            down_row, down_col, down_vals, down_bias)
  return out.reshape(shape)


# speedup vs baseline: 2.2473x; 2.2473x over previous
import functools

import jax
import jax.numpy as jnp
from jax import lax
from jax.experimental import pallas as pl
from jax.experimental.pallas import tpu as pltpu
from jax.experimental.pallas import tpu_sc as plsc

D = 1024
H = 4096
NNZ = 8192
B = 2048
W = 4
L = 16
NWORKERS = 32
SPT = B // (W * NWORKERS)
SHIFT = 13
CMASK = (1 << SHIFT) - 1


def _body(x_hbm, up_hbm, uv_hbm, ub_hbm,
          gp_hbm, gv_hbm, gb_hbm,
          dp_hbm, dv_hbm, db_hbm,
          out_hbm,
          x_v, up_acc, gate_acc, down_acc,
          up_v, uv_v, gp_v, gv_v, dp_v, dv_v,
          ub_v, gb_v, db_v):
  wid = lax.axis_index("s") * 2 + lax.axis_index("c")

  pltpu.sync_copy(ub_hbm, ub_v)
  pltpu.sync_copy(gb_hbm, gb_v)
  pltpu.sync_copy(db_hbm, db_v)
  pltpu.sync_copy(up_hbm, up_v)
  pltpu.sync_copy(uv_hbm, uv_v)
  pltpu.sync_copy(gp_hbm, gp_v)
  pltpu.sync_copy(gv_hbm, gv_v)
  pltpu.sync_copy(dp_hbm, dp_v)
  pltpu.sync_copy(dv_hbm, dv_v)

  def spmv(pk_v, val_v, src_ref, src_stride, acc_ref, acc_stride):
    @plsc.parallel_loop(0, NNZ // L, unroll=4)
    def _(g):
      o = g * L
      pk = pk_v[pl.ds(o, L)]
      vals = val_v[pl.ds(o, L)]
      rows = lax.shift_right_logical(pk, SHIFT)
      cols = lax.bitwise_and(pk, CMASK)
      for j in range(W):
        xg = plsc.load_gather(src_ref.at[pl.ds(j * src_stride, src_stride)],
                              [cols])
        plsc.addupdate_scatter(acc_ref.at[pl.ds(j * acc_stride, acc_stride)],
                               [rows], xg * vals)

  def slice_body(i, carry):
    sl = wid * SPT + i
    xoff = sl * (W * D)
    pltpu.sync_copy(x_hbm.at[pl.ds(xoff, W * D)], x_v)

    @plsc.parallel_loop(0, H // L, unroll=4)
    def _(k):
      o = k * L
      ub = ub_v[pl.ds(o, L)]
      gb = gb_v[pl.ds(o, L)]
      for j in range(W):
        up_acc[pl.ds(j * H + o, L)] = ub
        gate_acc[pl.ds(j * H + o, L)] = gb

    @plsc.parallel_loop(0, D // L, unroll=4)
    def _(k):
      o = k * L
      db = db_v[pl.ds(o, L)]
      for j in range(W):
        down_acc[pl.ds(j * D + o, L)] = db + x_v[pl.ds(j * D + o, L)]

    spmv(up_v, uv_v, x_v, D, up_acc, H)
    spmv(gp_v, gv_v, x_v, D, gate_acc, H)

    @plsc.parallel_loop(0, (W * H) // L, unroll=4)
    def _(k):
      o = k * L
      u = up_acc[pl.ds(o, L)]
      g = gate_acc[pl.ds(o, L)]
      up_acc[pl.ds(o, L)] = (u / (1.0 + jnp.exp(-u))) * g

    spmv(dp_v, dv_v, up_acc, H, down_acc, D)

    pltpu.sync_copy(down_acc, out_hbm.at[pl.ds(xoff, W * D)])
    return carry

  lax.fori_loop(0, SPT, slice_body, 0)


_sswiglu = functools.partial(
    pl.kernel,
    mesh=plsc.VectorSubcoreMesh(core_axis_name="c", subcore_axis_name="s"),
    out_type=jax.ShapeDtypeStruct((B * D,), jnp.float32),
    compiler_params=pltpu.CompilerParams(needs_layout_passes=False),
    scratch_types=[
        pltpu.VMEM((W * D,), jnp.float32),
        pltpu.VMEM((W * H,), jnp.float32),
        pltpu.VMEM((W * H,), jnp.float32),
        pltpu.VMEM((W * D,), jnp.float32),
        pltpu.VMEM((NNZ,), jnp.int32),
        pltpu.VMEM((NNZ,), jnp.float32),
        pltpu.VMEM((NNZ,), jnp.int32),
        pltpu.VMEM((NNZ,), jnp.float32),
        pltpu.VMEM((NNZ,), jnp.int32),
        pltpu.VMEM((NNZ,), jnp.float32),
        pltpu.VMEM((H,), jnp.float32),
        pltpu.VMEM((H,), jnp.float32),
        pltpu.VMEM((D,), jnp.float32),
    ],
)(_body)


def _pack(row, col):
  return jnp.bitwise_or(jnp.left_shift(row.astype(jnp.int32), SHIFT),
                        col.astype(jnp.int32))


def kernel(x, up_row, up_col, up_vals, up_bias,
           gate_row, gate_col, gate_vals, gate_bias,
           down_row, down_col, down_vals, down_bias):
  shape = x.shape
  out = _sswiglu(x.reshape(-1),
                 _pack(up_row, up_col), up_vals, up_bias,
                 _pack(gate_row, gate_col), gate_vals, gate_bias,
                 _pack(down_row, down_col), down_vals, down_bias)
  return out.reshape(shape)


# speedup vs baseline: 2.2494x; 1.0009x over previous
import functools

import jax
import jax.numpy as jnp
from jax import lax
from jax.experimental import pallas as pl
from jax.experimental.pallas import tpu as pltpu
from jax.experimental.pallas import tpu_sc as plsc

D = 1024
H = 4096
NNZ = 8192
B = 2048
W = 4
L = 16
NWORKERS = 32
SPT = B // (W * NWORKERS)
SHIFT = 13
CMASK = (1 << SHIFT) - 1


def _body(x_hbm, up_hbm, uv_hbm, ub_hbm,
          gp_hbm, gv_hbm, gb_hbm,
          dp_hbm, dv_hbm, db_hbm,
          out_hbm,
          x_v, up_acc, gate_acc, down_acc,
          up_v, uv_v, gp_v, gv_v, dp_v, dv_v,
          ub_v, gb_v, db_v):
  wid = lax.axis_index("s") * 2 + lax.axis_index("c")

  pltpu.sync_copy(ub_hbm, ub_v)
  pltpu.sync_copy(gb_hbm, gb_v)
  pltpu.sync_copy(db_hbm, db_v)
  pltpu.sync_copy(up_hbm, up_v)
  pltpu.sync_copy(uv_hbm, uv_v)
  pltpu.sync_copy(gp_hbm, gp_v)
  pltpu.sync_copy(gv_hbm, gv_v)
  pltpu.sync_copy(dp_hbm, dp_v)
  pltpu.sync_copy(dv_hbm, dv_v)

  def spmv(pk_v, val_v, src_ref, src_stride, acc_ref, acc_stride):
    @plsc.parallel_loop(0, NNZ // L, unroll=8)
    def _(g):
      o = g * L
      pk = pk_v[pl.ds(o, L)]
      vals = val_v[pl.ds(o, L)]
      rows = lax.shift_right_logical(pk, SHIFT)
      cols = lax.bitwise_and(pk, CMASK)
      for j in range(W):
        xg = plsc.load_gather(src_ref.at[pl.ds(j * src_stride, src_stride)],
                              [cols])
        plsc.addupdate_scatter(acc_ref.at[pl.ds(j * acc_stride, acc_stride)],
                               [rows], xg * vals)

  def slice_body(i, carry):
    sl = wid * SPT + i
    xoff = sl * (W * D)
    pltpu.sync_copy(x_hbm.at[pl.ds(xoff, W * D)], x_v)

    @plsc.parallel_loop(0, H // L, unroll=4)
    def _(k):
      o = k * L
      ub = ub_v[pl.ds(o, L)]
      gb = gb_v[pl.ds(o, L)]
      for j in range(W):
        up_acc[pl.ds(j * H + o, L)] = ub
        gate_acc[pl.ds(j * H + o, L)] = gb

    @plsc.parallel_loop(0, D // L, unroll=4)
    def _(k):
      o = k * L
      db = db_v[pl.ds(o, L)]
      for j in range(W):
        down_acc[pl.ds(j * D + o, L)] = db + x_v[pl.ds(j * D + o, L)]

    spmv(up_v, uv_v, x_v, D, up_acc, H)
    spmv(gp_v, gv_v, x_v, D, gate_acc, H)

    @plsc.parallel_loop(0, (W * H) // L, unroll=4)
    def _(k):
      o = k * L
      u = up_acc[pl.ds(o, L)]
      g = gate_acc[pl.ds(o, L)]
      up_acc[pl.ds(o, L)] = (u / (1.0 + jnp.exp(-u))) * g

    spmv(dp_v, dv_v, up_acc, H, down_acc, D)

    pltpu.sync_copy(down_acc, out_hbm.at[pl.ds(xoff, W * D)])
    return carry

  lax.fori_loop(0, SPT, slice_body, 0)


_sswiglu = functools.partial(
    pl.kernel,
    mesh=plsc.VectorSubcoreMesh(core_axis_name="c", subcore_axis_name="s"),
    out_type=jax.ShapeDtypeStruct((B * D,), jnp.float32),
    compiler_params=pltpu.CompilerParams(needs_layout_passes=False),
    scratch_types=[
        pltpu.VMEM((W * D,), jnp.float32),
        pltpu.VMEM((W * H,), jnp.float32),
        pltpu.VMEM((W * H,), jnp.float32),
        pltpu.VMEM((W * D,), jnp.float32),
        pltpu.VMEM((NNZ,), jnp.int32),
        pltpu.VMEM((NNZ,), jnp.float32),
        pltpu.VMEM((NNZ,), jnp.int32),
        pltpu.VMEM((NNZ,), jnp.float32),
        pltpu.VMEM((NNZ,), jnp.int32),
        pltpu.VMEM((NNZ,), jnp.float32),
        pltpu.VMEM((H,), jnp.float32),
        pltpu.VMEM((H,), jnp.float32),
        pltpu.VMEM((D,), jnp.float32),
    ],
)(_body)


def _pack(row, col):
  return jnp.bitwise_or(jnp.left_shift(row.astype(jnp.int32), SHIFT),
                        col.astype(jnp.int32))


def kernel(x, up_row, up_col, up_vals, up_bias,
           gate_row, gate_col, gate_vals, gate_bias,
           down_row, down_col, down_vals, down_bias):
  shape = x.shape
  out = _sswiglu(x.reshape(-1),
                 _pack(up_row, up_col), up_vals, up_bias,
                 _pack(gate_row, gate_col), gate_vals, gate_bias,
                 _pack(down_row, down_col), down_vals, down_bias)
  return out.reshape(shape)


# speedup vs baseline: 2.3614x; 1.0498x over previous
import functools

import jax
import jax.numpy as jnp
from jax import lax
from jax.experimental import pallas as pl
from jax.experimental.pallas import tpu as pltpu
from jax.experimental.pallas import tpu_sc as plsc

D = 1024
H = 4096
NNZ = 8192
B = 2048
W = 4
L = 16
NWORKERS = 32
SPT = B // (W * NWORKERS)
SHIFT = 13
CMASK = (1 << SHIFT) - 1


def _body(x_hbm, up_hbm, uv_hbm, ub_hbm,
          gp_hbm, gv_hbm, gb_hbm,
          dp_hbm, dv_hbm, db_hbm,
          out_hbm,
          x_vs, up_acc, gate_acc, out_vs,
          up_v, uv_v, gp_v, gv_v, dp_v, dv_v,
          ub_v, gb_v, db_v,
          sem_ins, sem_outs, sem_stage):
  wid = lax.axis_index("s") * 2 + lax.axis_index("c")
  base = wid * SPT

  def x_copy(s, b):
    return pltpu.make_async_copy(
        x_hbm.at[pl.ds(s * (W * D), W * D)], x_vs[b], sem_ins[b])

  def out_copy(s, b):
    return pltpu.make_async_copy(
        out_vs[b], out_hbm.at[pl.ds(s * (W * D), W * D)], sem_outs[b])

  x_copy(base, 0).start()
  stage = [pltpu.async_copy(s, d, sem_stage) for s, d in (
      (ub_hbm, ub_v), (gb_hbm, gb_v), (db_hbm, db_v),
      (up_hbm, up_v), (uv_hbm, uv_v),
      (gp_hbm, gp_v), (gv_hbm, gv_v),
      (dp_hbm, dp_v), (dv_hbm, dv_v))]
  for c in stage:
    c.wait()

  def spmv(pk_v, val_v, src_ref, src_stride, acc_ref, acc_stride):
    @plsc.parallel_loop(0, NNZ // L, unroll=8)
    def _(g):
      o = g * L
      pk = pk_v[pl.ds(o, L)]
      vals = val_v[pl.ds(o, L)]
      rows = lax.shift_right_logical(pk, SHIFT)
      cols = lax.bitwise_and(pk, CMASK)
      for j in range(W):
        xg = plsc.load_gather(src_ref.at[pl.ds(j * src_stride, src_stride)],
                              [cols])
        plsc.addupdate_scatter(acc_ref.at[pl.ds(j * acc_stride, acc_stride)],
                               [rows], xg * vals)

  def pair_body(kk, carry):
    for b in range(2):
      sloc = kk * 2 + b
      s = base + sloc
      x_v = x_vs[b]
      out_v = out_vs[b]

      x_copy(s, b).wait()
      if b == 0:
        x_copy(s + 1, 1).start()
      else:
        @pl.when(kk < SPT // 2 - 1)
        def _():
          x_copy(s + 1, 0).start()

      @plsc.parallel_loop(0, H // L, unroll=4)
      def _(k):
        o = k * L
        ub = ub_v[pl.ds(o, L)]
        gb = gb_v[pl.ds(o, L)]
        for j in range(W):
          up_acc[pl.ds(j * H + o, L)] = ub
          gate_acc[pl.ds(j * H + o, L)] = gb

      @pl.when(kk > 0)
      def _():
        out_copy(s - 2, b).wait()

      @plsc.parallel_loop(0, D // L, unroll=4)
      def _(k):
        o = k * L
        db = db_v[pl.ds(o, L)]
        for j in range(W):
          out_v[pl.ds(j * D + o, L)] = db + x_v[pl.ds(j * D + o, L)]

      spmv(up_v, uv_v, x_v, D, up_acc, H)
      spmv(gp_v, gv_v, x_v, D, gate_acc, H)

      @plsc.parallel_loop(0, (W * H) // L, unroll=4)
      def _(k):
        o = k * L
        u = up_acc[pl.ds(o, L)]
        g = gate_acc[pl.ds(o, L)]
        up_acc[pl.ds(o, L)] = (u / (1.0 + jnp.exp(-u))) * g

      spmv(dp_v, dv_v, up_acc, H, out_v, D)

      out_copy(s, b).start()
    return carry

  lax.fori_loop(0, SPT // 2, pair_body, 0)

  out_copy(base + SPT - 2, 0).wait()
  out_copy(base + SPT - 1, 1).wait()


_sswiglu = functools.partial(
    pl.kernel,
    mesh=plsc.VectorSubcoreMesh(core_axis_name="c", subcore_axis_name="s"),
    out_type=jax.ShapeDtypeStruct((B * D,), jnp.float32),
    compiler_params=pltpu.CompilerParams(needs_layout_passes=False),
    scratch_types=[
        [pltpu.VMEM((W * D,), jnp.float32)] * 2,
        pltpu.VMEM((W * H,), jnp.float32),
        pltpu.VMEM((W * H,), jnp.float32),
        [pltpu.VMEM((W * D,), jnp.float32)] * 2,
        pltpu.VMEM((NNZ,), jnp.int32),
        pltpu.VMEM((NNZ,), jnp.float32),
        pltpu.VMEM((NNZ,), jnp.int32),
        pltpu.VMEM((NNZ,), jnp.float32),
        pltpu.VMEM((NNZ,), jnp.int32),
        pltpu.VMEM((NNZ,), jnp.float32),
        pltpu.VMEM((H,), jnp.float32),
        pltpu.VMEM((H,), jnp.float32),
        pltpu.VMEM((D,), jnp.float32),
        [pltpu.SemaphoreType.DMA] * 2,
        [pltpu.SemaphoreType.DMA] * 2,
        pltpu.SemaphoreType.DMA,
    ],
)(_body)


def _pack(row, col):
  return jnp.bitwise_or(jnp.left_shift(row.astype(jnp.int32), SHIFT),
                        col.astype(jnp.int32))


def kernel(x, up_row, up_col, up_vals, up_bias,
           gate_row, gate_col, gate_vals, gate_bias,
           down_row, down_col, down_vals, down_bias):
  shape = x.shape
  out = _sswiglu(x.reshape(-1),
                 _pack(up_row, up_col), up_vals, up_bias,
                 _pack(gate_row, gate_col), gate_vals, gate_bias,
                 _pack(down_row, down_col), down_vals, down_bias)
  return out.reshape(shape)


# speedup vs baseline: 2.5175x; 1.0661x over previous
import functools

import jax
import jax.numpy as jnp
from jax import lax
from jax.experimental import pallas as pl
from jax.experimental.pallas import tpu as pltpu
from jax.experimental.pallas import tpu_sc as plsc

D = 1024
H = 4096
NNZ = 8192
B = 2048
W = 4
L = 16
NWORKERS = 32
SPT = B // (W * NWORKERS)
SHIFT = 13
CMASK = (1 << SHIFT) - 1


def _body(x_hbm, up_hbm, uv_hbm, ub_hbm,
          gp_hbm, gv_hbm, gb_hbm,
          dp_hbm, dv_hbm, db_hbm,
          out_hbm,
          x_vs, up_acc, gate_acc, out_vs,
          up_v, uv_v, gp_v, gv_v, dp_v, dv_v,
          ub_v, gb_v, db_v,
          sem_ins, sem_outs, sem_stage):
  wid = lax.axis_index("s") * 2 + lax.axis_index("c")
  base = wid * SPT

  def x_copy(s, b):
    return pltpu.make_async_copy(
        x_hbm.at[pl.ds(s * W, W), :], x_vs[b], sem_ins[b])

  def out_copy(s, b):
    return pltpu.make_async_copy(
        out_vs[b], out_hbm.at[pl.ds(s * W, W), :], sem_outs[b])

  x_copy(base, 0).start()
  stage = [pltpu.async_copy(s, d, sem_stage) for s, d in (
      (ub_hbm, ub_v), (gb_hbm, gb_v), (db_hbm, db_v),
      (up_hbm, up_v), (uv_hbm, uv_v),
      (gp_hbm, gp_v), (gv_hbm, gv_v),
      (dp_hbm, dp_v), (dv_hbm, dv_v))]
  for c in stage:
    c.wait()

  def spmv(pk_v, val_v, src_idx, acc_idx):
    @plsc.parallel_loop(0, NNZ // L, unroll=8)
    def _(g):
      o = g * L
      pk = pk_v[pl.ds(o, L)]
      vals = val_v[pl.ds(o, L)]
      rows = lax.shift_right_logical(pk, SHIFT)
      cols = lax.bitwise_and(pk, CMASK)
      for j in range(W):
        ref, idx = src_idx(j, cols)
        xg = plsc.load_gather(ref, idx)
        ref, idx = acc_idx(j, rows)
        plsc.addupdate_scatter(ref, idx, xg * vals)

  def pair_body(kk, carry):
    for b in range(2):
      sloc = kk * 2 + b
      s = base + sloc
      x_v = x_vs[b]
      out_v = out_vs[b]

      x_copy(s, b).wait()
      if b == 0:
        x_copy(s + 1, 1).start()
      else:
        @pl.when(kk < SPT // 2 - 1)
        def _():
          x_copy(s + 1, 0).start()

      @plsc.parallel_loop(0, H // L, unroll=4)
      def _(k):
        o = k * L
        ub = ub_v[pl.ds(o, L)]
        gb = gb_v[pl.ds(o, L)]
        for j in range(W):
          up_acc[pl.ds(j * H + o, L)] = ub
          gate_acc[pl.ds(j * H + o, L)] = gb

      @pl.when(kk > 0)
      def _():
        out_copy(s - 2, b).wait()

      @plsc.parallel_loop(0, D // L, unroll=4)
      def _(k):
        o = k * L
        db = db_v[pl.ds(o, L)]
        for j in range(W):
          out_v[j, pl.ds(o, L)] = db + x_v[j, pl.ds(o, L)]

      def row_of(j):
        return jnp.full((L,), j, jnp.int32)

      def x_idx(j, cols):
        return x_v, [row_of(j), cols]

      def out_idx(j, rows):
        return out_v, [row_of(j), rows]

      def up_idx(j, rows):
        return up_acc.at[pl.ds(j * H, H)], [rows]

      def gate_idx(j, rows):
        return gate_acc.at[pl.ds(j * H, H)], [rows]

      spmv(up_v, uv_v, x_idx, up_idx)
      spmv(gp_v, gv_v, x_idx, gate_idx)

      @plsc.parallel_loop(0, (W * H) // L, unroll=4)
      def _(k):
        o = k * L
        u = up_acc[pl.ds(o, L)]
        g = gate_acc[pl.ds(o, L)]
        up_acc[pl.ds(o, L)] = (u / (1.0 + jnp.exp(-u))) * g

      spmv(dp_v, dv_v, lambda j, cols: (up_acc.at[pl.ds(j * H, H)], [cols]),
           out_idx)

      out_copy(s, b).start()
    return carry

  lax.fori_loop(0, SPT // 2, pair_body, 0)

  out_copy(base + SPT - 2, 0).wait()
  out_copy(base + SPT - 1, 1).wait()


_sswiglu = functools.partial(
    pl.kernel,
    mesh=plsc.VectorSubcoreMesh(core_axis_name="c", subcore_axis_name="s"),
    out_type=jax.ShapeDtypeStruct((B, D), jnp.float32),
    compiler_params=pltpu.CompilerParams(needs_layout_passes=False),
    scratch_types=[
        [pltpu.VMEM((W, D), jnp.float32)] * 2,
        pltpu.VMEM((W * H,), jnp.float32),
        pltpu.VMEM((W * H,), jnp.float32),
        [pltpu.VMEM((W, D), jnp.float32)] * 2,
        pltpu.VMEM((NNZ,), jnp.int32),
        pltpu.VMEM((NNZ,), jnp.float32),
        pltpu.VMEM((NNZ,), jnp.int32),
        pltpu.VMEM((NNZ,), jnp.float32),
        pltpu.VMEM((NNZ,), jnp.int32),
        pltpu.VMEM((NNZ,), jnp.float32),
        pltpu.VMEM((H,), jnp.float32),
        pltpu.VMEM((H,), jnp.float32),
        pltpu.VMEM((D,), jnp.float32),
        [pltpu.SemaphoreType.DMA] * 2,
        [pltpu.SemaphoreType.DMA] * 2,
        pltpu.SemaphoreType.DMA,
    ],
)(_body)


def _pack(row, col):
  return jnp.bitwise_or(jnp.left_shift(row.astype(jnp.int32), SHIFT),
                        col.astype(jnp.int32))


def kernel(x, up_row, up_col, up_vals, up_bias,
           gate_row, gate_col, gate_vals, gate_bias,
           down_row, down_col, down_vals, down_bias):
  shape = x.shape
  out = _sswiglu(x.reshape(B, D),
                 _pack(up_row, up_col), up_vals, up_bias,
                 _pack(gate_row, gate_col), gate_vals, gate_bias,
                 _pack(down_row, down_col), down_vals, down_bias)
  return out.reshape(shape)


# speedup vs baseline: 2.5751x; 1.0229x over previous
import functools

import jax
import jax.numpy as jnp
from jax import lax
from jax.experimental import pallas as pl
from jax.experimental.pallas import tpu as pltpu
from jax.experimental.pallas import tpu_sc as plsc

D = 1024
H = 4096
NNZ = 8192
B = 2048
W = 4
L = 16
NWORKERS = 32
SPT = B // (W * NWORKERS)
SHIFT = 13
CMASK = (1 << SHIFT) - 1


def _body(x_hbm, up_hbm, uv_hbm, ub_hbm,
          gp_hbm, gv_hbm, gb_hbm,
          dp_hbm, dv_hbm, db_hbm,
          out_hbm,
          x_vs, up_acc, gate_acc, out_vs,
          up_v, uv_v, gp_v, gv_v, dp_v, dv_v,
          ub_v, gb_v, db_v,
          sem_ins, sem_outs, sem_stage):
  wid = lax.axis_index("s") * 2 + lax.axis_index("c")
  base = wid * SPT

  def x_copy(s, b):
    return pltpu.make_async_copy(
        x_hbm.at[pl.ds(s * W, W), :], x_vs[b], sem_ins[b])

  def out_copy(s, b):
    return pltpu.make_async_copy(
        out_vs[b], out_hbm.at[pl.ds(s * W, W), :], sem_outs[b])

  x_copy(base, 0).start()
  stage = [pltpu.async_copy(s, d, sem_stage) for s, d in (
      (ub_hbm, ub_v), (gb_hbm, gb_v), (db_hbm, db_v),
      (up_hbm, up_v), (uv_hbm, uv_v),
      (gp_hbm, gp_v), (gv_hbm, gv_v),
      (dp_hbm, dp_v), (dv_hbm, dv_v))]
  for c in stage:
    c.wait()

  @plsc.parallel_loop(0, H // L, unroll=4)
  def _(k):
    o = k * L
    gb = gb_v[pl.ds(o, L)]
    for j in range(W):
      gate_acc[pl.ds(j * H + o, L)] = gb

  def spmv(pk_v, val_v, src_idx, acc_idx):
    @plsc.parallel_loop(0, NNZ // L, unroll=8)
    def _(g):
      o = g * L
      pk = pk_v[pl.ds(o, L)]
      vals = val_v[pl.ds(o, L)]
      rows = lax.shift_right_logical(pk, SHIFT)
      cols = lax.bitwise_and(pk, CMASK)
      for j in range(W):
        ref, idx = src_idx(j, cols)
        xg = plsc.load_gather(ref, idx)
        ref, idx = acc_idx(j, rows)
        plsc.addupdate_scatter(ref, idx, xg * vals)

  def pair_body(kk, carry):
    for b in range(2):
      sloc = kk * 2 + b
      s = base + sloc
      x_v = x_vs[b]
      out_v = out_vs[b]

      x_copy(s, b).wait()
      if b == 0:
        x_copy(s + 1, 1).start()
      else:
        @pl.when(kk < SPT // 2 - 1)
        def _():
          x_copy(s + 1, 0).start()

      @plsc.parallel_loop(0, H // L, unroll=4)
      def _(k):
        o = k * L
        ub = ub_v[pl.ds(o, L)]
        for j in range(W):
          up_acc[pl.ds(j * H + o, L)] = ub

      @pl.when(kk > 0)
      def _():
        out_copy(s - 2, b).wait()

      @plsc.parallel_loop(0, D // L, unroll=4)
      def _(k):
        o = k * L
        db = db_v[pl.ds(o, L)]
        for j in range(W):
          out_v[j, pl.ds(o, L)] = db + x_v[j, pl.ds(o, L)]

      def row_of(j):
        return jnp.full((L,), j, jnp.int32)

      def x_idx(j, cols):
        return x_v, [row_of(j), cols]

      def out_idx(j, rows):
        return out_v, [row_of(j), rows]

      def up_idx(j, rows):
        return up_acc.at[pl.ds(j * H, H)], [rows]

      def gate_idx(j, rows):
        return gate_acc.at[pl.ds(j * H, H)], [rows]

      spmv(up_v, uv_v, x_idx, up_idx)
      spmv(gp_v, gv_v, x_idx, gate_idx)

      @plsc.parallel_loop(0, H // L, unroll=4)
      def _(k):
        o = k * L
        gb = gb_v[pl.ds(o, L)]
        for j in range(W):
          u = up_acc[pl.ds(j * H + o, L)]
          g = gate_acc[pl.ds(j * H + o, L)]
          up_acc[pl.ds(j * H + o, L)] = (u / (1.0 + jnp.exp(-u))) * g
          gate_acc[pl.ds(j * H + o, L)] = gb

      spmv(dp_v, dv_v, lambda j, cols: (up_acc.at[pl.ds(j * H, H)], [cols]),
           out_idx)

      out_copy(s, b).start()
    return carry

  lax.fori_loop(0, SPT // 2, pair_body, 0)

  out_copy(base + SPT - 2, 0).wait()
  out_copy(base + SPT - 1, 1).wait()


_sswiglu = functools.partial(
    pl.kernel,
    mesh=plsc.VectorSubcoreMesh(core_axis_name="c", subcore_axis_name="s"),
    out_type=jax.ShapeDtypeStruct((B, D), jnp.float32),
    compiler_params=pltpu.CompilerParams(needs_layout_passes=False),
    scratch_types=[
        [pltpu.VMEM((W, D), jnp.float32)] * 2,
        pltpu.VMEM((W * H,), jnp.float32),
        pltpu.VMEM((W * H,), jnp.float32),
        [pltpu.VMEM((W, D), jnp.float32)] * 2,
        pltpu.VMEM((NNZ,), jnp.int32),
        pltpu.VMEM((NNZ,), jnp.float32),
        pltpu.VMEM((NNZ,), jnp.int32),
        pltpu.VMEM((NNZ,), jnp.float32),
        pltpu.VMEM((NNZ,), jnp.int32),
        pltpu.VMEM((NNZ,), jnp.float32),
        pltpu.VMEM((H,), jnp.float32),
        pltpu.VMEM((H,), jnp.float32),
        pltpu.VMEM((D,), jnp.float32),
        [pltpu.SemaphoreType.DMA] * 2,
        [pltpu.SemaphoreType.DMA] * 2,
        pltpu.SemaphoreType.DMA,
    ],
)(_body)


def _pack(row, col):
  return jnp.bitwise_or(jnp.left_shift(row.astype(jnp.int32), SHIFT),
                        col.astype(jnp.int32))


def kernel(x, up_row, up_col, up_vals, up_bias,
           gate_row, gate_col, gate_vals, gate_bias,
           down_row, down_col, down_vals, down_bias):
  shape = x.shape
  out = _sswiglu(x.reshape(B, D),
                 _pack(up_row, up_col), up_vals, up_bias,
                 _pack(gate_row, gate_col), gate_vals, gate_bias,
                 _pack(down_row, down_col), down_vals, down_bias)
  return out.reshape(shape)
